# baseline (speedup 1.0000x reference)
import functools

import jax
import jax.numpy as jnp
from jax import lax
from jax.experimental import pallas as pl
from jax.experimental.pallas import tpu as pltpu
from jax.experimental.pallas import tpu_sc as plsc

VOCAB = 1000000
EMBED_DIM = 64
NC, NS = 2, 16
NW = NC * NS
GRP = 128
NBUF = 8


def _make_kernel(n_rows):
    assert n_rows % (NW * GRP) == 0
    rows_per_w = n_rows // NW
    grps_per_w = rows_per_w // GRP
    n_iter = grps_per_w // NBUF
    assert grps_per_w % NBUF == 0

    mesh = plsc.VectorSubcoreMesh(core_axis_name="c", subcore_axis_name="s")

    @functools.partial(
        pl.kernel,
        out_type=jax.ShapeDtypeStruct((n_rows, EMBED_DIM), jnp.float32),
        mesh=mesh,
        compiler_params=pltpu.CompilerParams(use_tc_tiling_on_sc=False),
        scratch_types=[
            pltpu.VMEM((grps_per_w, GRP), jnp.int32),
            pltpu.VMEM((NBUF, GRP, EMBED_DIM), jnp.float32),
            pltpu.SemaphoreType.DMA,
            pltpu.SemaphoreType.DMA,
        ],
    )
    def emb(x_hbm, table_hbm, out_hbm, idx_v, rows_v, gsem, osem):
        wid = lax.axis_index("s") * NC + lax.axis_index("c")
        grp0 = wid * grps_per_w

        pltpu.sync_copy(x_hbm.at[pl.ds(grp0, grps_per_w)], idx_v)

        def body(i, carry):
            base_g = i * NBUF
            for j in range(NBUF):
                pltpu.make_async_copy(
                    table_hbm.at[idx_v.at[base_g + j]], rows_v.at[j], gsem
                ).start()
            for j in range(NBUF):
                pltpu.make_async_copy(
                    table_hbm.at[idx_v.at[base_g + j]], rows_v.at[j], gsem
                ).wait()
            for j in range(NBUF):
                row0 = (grp0 + base_g + j) * GRP
                pltpu.make_async_copy(
                    rows_v.at[j], out_hbm.at[pl.ds(row0, GRP)], osem
                ).start()
            for j in range(NBUF):
                row0 = (grp0 + base_g + j) * GRP
                pltpu.make_async_copy(
                    rows_v.at[j], out_hbm.at[pl.ds(row0, GRP)], osem
                ).wait()
            return carry

        lax.fori_loop(0, n_iter, body, 0)

    return emb


def kernel(x, table):
    b, s = x.shape
    n_rows = b * s
    x_flat = x.reshape(n_rows // GRP, GRP).astype(jnp.int32)
    out = _make_kernel(n_rows)(x_flat, table)
    return out.reshape(b, s, EMBED_DIM)


# speedup vs baseline: 1.0063x; 1.0063x over previous
import functools

import jax
import jax.numpy as jnp
from jax import lax
from jax.experimental import pallas as pl
from jax.experimental.pallas import tpu as pltpu
from jax.experimental.pallas import tpu_sc as plsc

VOCAB = 1000000
EMBED_DIM = 64
NC, NS = 2, 16
NW = NC * NS
GRP = 128
NBUF = 4


def _make_kernel(n_rows):
    rows_per_w = n_rows // NW
    grps_per_w = rows_per_w // GRP
    n_rounds = grps_per_w // NBUF
    n_pairs = n_rounds // 2
    assert n_rows % (NW * GRP) == 0 and grps_per_w % (2 * NBUF) == 0

    mesh = plsc.VectorSubcoreMesh(core_axis_name="c", subcore_axis_name="s")

    @functools.partial(
        pl.kernel,
        out_type=jax.ShapeDtypeStruct((n_rows, EMBED_DIM), jnp.float32),
        mesh=mesh,
        compiler_params=pltpu.CompilerParams(use_tc_tiling_on_sc=False),
        scratch_types=[
            pltpu.VMEM((grps_per_w, GRP), jnp.int32),
            pltpu.VMEM((2, NBUF, GRP, EMBED_DIM), jnp.float32),
            pltpu.SemaphoreType.DMA,
            pltpu.SemaphoreType.DMA,
            pltpu.SemaphoreType.DMA,
            pltpu.SemaphoreType.DMA,
        ],
    )
    def emb(x_hbm, table_hbm, out_hbm, idx_v, rows_v, gsa, gsb, wsa, wsb):
        wid = lax.axis_index("s") * NC + lax.axis_index("c")
        grp0 = wid * grps_per_w
        gsem = (gsa, gsb)
        wsem = (wsa, wsb)

        pltpu.sync_copy(x_hbm.at[pl.ds(grp0, grps_per_w)], idx_v)

        def fire_gathers(r, h):
            for j in range(NBUF):
                g = r * NBUF + j
                pltpu.make_async_copy(
                    table_hbm.at[idx_v.at[g]], rows_v.at[h, j], gsem[h]
                ).start()

        def drain_gathers(r, h):
            for j in range(NBUF):
                g = r * NBUF + j
                pltpu.make_async_copy(
                    table_hbm.at[idx_v.at[g]], rows_v.at[h, j], gsem[h]
                ).wait()

        def fire_writes(r, h):
            for j in range(NBUF):
                row0 = (grp0 + r * NBUF + j) * GRP
                pltpu.make_async_copy(
                    rows_v.at[h, j], out_hbm.at[pl.ds(row0, GRP)], wsem[h]
                ).start()

        def drain_writes(r, h):
            for j in range(NBUF):
                row0 = (grp0 + r * NBUF + j) * GRP
                pltpu.make_async_copy(
                    rows_v.at[h, j], out_hbm.at[pl.ds(row0, GRP)], wsem[h]
                ).wait()

        fire_gathers(0, 0)
        fire_gathers(1, 1)

        def body(i, carry):
            r0 = 2 * i
            r1 = 2 * i + 1
            drain_gathers(r0, 0)
            fire_writes(r0, 0)
            drain_gathers(r1, 1)
            fire_writes(r1, 1)
            drain_writes(r0, 0)
            fire_gathers(r0 + 2, 0)
            drain_writes(r1, 1)
            fire_gathers(r1 + 2, 1)
            return carry

        lax.fori_loop(0, n_pairs - 1, body, 0)

        rl0, rl1 = n_rounds - 2, n_rounds - 1
        drain_gathers(rl0, 0)
        fire_writes(rl0, 0)
        drain_gathers(rl1, 1)
        fire_writes(rl1, 1)
        drain_writes(rl0, 0)
        drain_writes(rl1, 1)

    return emb


def kernel(x, table):
    b, s = x.shape
    n_rows = b * s
    x_flat = x.reshape(n_rows // GRP, GRP).astype(jnp.int32)
    out = _make_kernel(n_rows)(x_flat, table)
    return out.reshape(b, s, EMBED_DIM)


# speedup vs baseline: 1.0248x; 1.0184x over previous
import functools

import jax
import jax.numpy as jnp
from jax import lax
from jax.experimental import pallas as pl
from jax.experimental.pallas import tpu as pltpu
from jax.experimental.pallas import tpu_sc as plsc

VOCAB = 1000000
EMBED_DIM = 64
NC, NS = 2, 16
NW = NC * NS
GRP = 128
NBUF = 4
TBLK = 1024
NBLK = -(-VOCAB // TBLK)
REC_ROWS = NBLK * TBLK


def _make_records():
    def rec(tab_ref, out_ref):
        out_ref[:, pl.ds(0, EMBED_DIM)] = tab_ref[...].T

    return pl.pallas_call(
        rec,
        grid=(NBLK,),
        in_specs=[pl.BlockSpec((EMBED_DIM, TBLK), lambda i: (0, i))],
        out_specs=pl.BlockSpec((TBLK, 128), lambda i: (i, 0)),
        out_shape=jax.ShapeDtypeStruct((REC_ROWS, 128), jnp.float32),
    )


def _make_gather(n_rows):
    rows_per_w = n_rows // NW
    grps_per_w = rows_per_w // GRP
    n_iter = grps_per_w // NBUF
    assert n_rows % (NW * GRP) == 0 and grps_per_w % NBUF == 0

    mesh = plsc.VectorSubcoreMesh(core_axis_name="c", subcore_axis_name="s")

    @functools.partial(
        pl.kernel,
        out_type=jax.ShapeDtypeStruct((n_rows, 128), jnp.float32),
        mesh=mesh,
        scratch_types=[
            pltpu.VMEM((grps_per_w, GRP), jnp.int32),
            pltpu.VMEM((NBUF, GRP, 128), jnp.float32),
            pltpu.SemaphoreType.DMA,
            pltpu.SemaphoreType.DMA,
        ],
    )
    def emb(x_hbm, rec_hbm, out_hbm, idx_v, rows_v, gsem, osem):
        wid = lax.axis_index("s") * NC + lax.axis_index("c")
        grp0 = wid * grps_per_w

        pltpu.sync_copy(x_hbm.at[pl.ds(grp0, grps_per_w)], idx_v)

        def gcopy(g, j):
            return pltpu.make_async_copy(
                rec_hbm.at[idx_v.at[g]], rows_v.at[j], gsem
            )

        def ocopy(g, j):
            return pltpu.make_async_copy(
                rows_v.at[j], out_hbm.at[pl.ds((grp0 + g) * GRP, GRP)], osem
            )

        def body(i, carry):
            base = i * NBUF
            for j in range(NBUF):
                gcopy(base + j, j).start()
            for j in range(NBUF):
                gcopy(base + j, j).wait()
            for j in range(NBUF):
                ocopy(base + j, j).start()
            for j in range(NBUF):
                ocopy(base + j, j).wait()
            return carry

        lax.fori_loop(0, n_iter, body, 0)

    return emb


def kernel(x, table):
    b, s = x.shape
    n_rows = b * s
    x_flat = x.reshape(n_rows // GRP, GRP).astype(jnp.int32)
    records = _make_records()(table.T)
    out = _make_gather(n_rows)(x_flat, records)
    return out.reshape(b, s, 128)[:, :, :EMBED_DIM]


# speedup vs baseline: 1.2595x; 1.2290x over previous
import functools

import jax
import jax.numpy as jnp
from jax import lax
from jax.experimental import pallas as pl
from jax.experimental.pallas import tpu as pltpu
from jax.experimental.pallas import tpu_sc as plsc

VOCAB = 1000000
EMBED_DIM = 64
NC, NS = 2, 16
NW = NC * NS
GRP = 128
NBUF = 4
TBLK = 4096
NBLK = -(-VOCAB // TBLK)
REC_ROWS = NBLK * TBLK


def _make_records():
    def rec(tab_ref, out_ref):
        i0 = lax.broadcasted_iota(jnp.int32, (EMBED_DIM, EMBED_DIM), 0)
        i1 = lax.broadcasted_iota(jnp.int32, (EMBED_DIM, EMBED_DIM), 1)
        eye = (i0 == i1).astype(jnp.float32)
        out_ref[:, pl.ds(0, EMBED_DIM)] = lax.dot_general(
            tab_ref[...], eye, (((0,), (0,)), ((), ())),
            preferred_element_type=jnp.float32,
            precision=lax.Precision.HIGHEST,
        )

    return pl.pallas_call(
        rec,
        grid=(NBLK,),
        in_specs=[pl.BlockSpec((EMBED_DIM, TBLK), lambda i: (0, i))],
        out_specs=pl.BlockSpec((TBLK, 128), lambda i: (i, 0)),
        out_shape=jax.ShapeDtypeStruct((REC_ROWS, 128), jnp.float32),
    )


def _make_gather(n_rows):
    rows_per_w = n_rows // NW
    grps_per_w = rows_per_w // GRP
    n_iter = grps_per_w // NBUF
    assert n_rows % (NW * GRP) == 0 and grps_per_w % NBUF == 0

    mesh = plsc.VectorSubcoreMesh(core_axis_name="c", subcore_axis_name="s")

    @functools.partial(
        pl.kernel,
        out_type=jax.ShapeDtypeStruct((n_rows, 128), jnp.float32),
        mesh=mesh,
        scratch_types=[
            pltpu.VMEM((grps_per_w, GRP), jnp.int32),
            pltpu.VMEM((NBUF, GRP, 128), jnp.float32),
            pltpu.SemaphoreType.DMA,
            pltpu.SemaphoreType.DMA,
        ],
    )
    def emb(x_hbm, rec_hbm, out_hbm, idx_v, rows_v, gsem, osem):
        wid = lax.axis_index("s") * NC + lax.axis_index("c")
        grp0 = wid * grps_per_w

        pltpu.sync_copy(x_hbm.at[pl.ds(grp0, grps_per_w)], idx_v)

        def gcopy(g, j):
            return pltpu.make_async_copy(
                rec_hbm.at[idx_v.at[g]], rows_v.at[j], gsem
            )

        def ocopy(g, j):
            return pltpu.make_async_copy(
                rows_v.at[j], out_hbm.at[pl.ds((grp0 + g) * GRP, GRP)], osem
            )

        def body(i, carry):
            base = i * NBUF
            for j in range(NBUF):
                gcopy(base + j, j).start()
            for j in range(NBUF):
                gcopy(base + j, j).wait()
            for j in range(NBUF):
                ocopy(base + j, j).start()
            for j in range(NBUF):
                ocopy(base + j, j).wait()
            return carry

        lax.fori_loop(0, n_iter, body, 0)

    return emb


def kernel(x, table):
    b, s = x.shape
    n_rows = b * s
    x_flat = x.reshape(n_rows // GRP, GRP).astype(jnp.int32)
    records = _make_records()(table.T)
    out = _make_gather(n_rows)(x_flat, records)
    return out.reshape(b, s, 128)[:, :, :EMBED_DIM]


# speedup vs baseline: 1.2630x; 1.0028x over previous
import functools

import jax
import jax.numpy as jnp
from jax import lax
from jax.experimental import pallas as pl
from jax.experimental.pallas import tpu as pltpu
from jax.experimental.pallas import tpu_sc as plsc

VOCAB = 1000000
EMBED_DIM = 64
NC, NS = 2, 16
NW = NC * NS
GRP = 128
NBUF = 4
TBLK = 4096
NBLK = -(-VOCAB // TBLK)
REC_ROWS = NBLK * TBLK


def _make_records():
    def rec(tab_ref, out_ref):
        i0 = lax.broadcasted_iota(jnp.int32, (EMBED_DIM, 128), 0)
        i1 = lax.broadcasted_iota(jnp.int32, (EMBED_DIM, 128), 1)
        eye = (i0 == i1).astype(jnp.float32)
        out_ref[...] = lax.dot_general(
            tab_ref[...], eye, (((0,), (0,)), ((), ())),
            preferred_element_type=jnp.float32,
            precision=lax.Precision.HIGHEST,
        )

    return pl.pallas_call(
        rec,
        grid=(NBLK,),
        in_specs=[pl.BlockSpec((EMBED_DIM, TBLK), lambda i: (0, i))],
        out_specs=pl.BlockSpec((TBLK, 128), lambda i: (i, 0)),
        out_shape=jax.ShapeDtypeStruct((REC_ROWS, 128), jnp.float32),
    )


def _make_gather(n_rows):
    rows_per_w = n_rows // NW
    grps_per_w = rows_per_w // GRP
    n_iter = grps_per_w // NBUF
    assert n_rows % (NW * GRP) == 0 and grps_per_w % NBUF == 0

    mesh = plsc.VectorSubcoreMesh(core_axis_name="c", subcore_axis_name="s")

    @functools.partial(
        pl.kernel,
        out_type=jax.ShapeDtypeStruct((n_rows, 128), jnp.float32),
        mesh=mesh,
        scratch_types=[
            pltpu.VMEM((grps_per_w, GRP), jnp.int32),
            pltpu.VMEM((NBUF, GRP, 128), jnp.float32),
            pltpu.SemaphoreType.DMA,
            pltpu.SemaphoreType.DMA,
        ],
    )
    def emb(x_hbm, rec_hbm, out_hbm, idx_v, rows_v, gsem, osem):
        wid = lax.axis_index("s") * NC + lax.axis_index("c")
        grp0 = wid * grps_per_w

        pltpu.sync_copy(x_hbm.at[pl.ds(grp0, grps_per_w)], idx_v)

        def gcopy(g, j):
            return pltpu.make_async_copy(
                rec_hbm.at[idx_v.at[g]], rows_v.at[j], gsem
            )

        def ocopy(g, j):
            return pltpu.make_async_copy(
                rows_v.at[j], out_hbm.at[pl.ds((grp0 + g) * GRP, GRP)], osem
            )

        def body(i, carry):
            base = i * NBUF
            for j in range(NBUF):
                gcopy(base + j, j).start()
            for j in range(NBUF):
                gcopy(base + j, j).wait()
            for j in range(NBUF):
                ocopy(base + j, j).start()
            for j in range(NBUF):
                ocopy(base + j, j).wait()
            return carry

        lax.fori_loop(0, n_iter, body, 0)

    return emb


def kernel(x, table):
    b, s = x.shape
    n_rows = b * s
    x_flat = x.reshape(n_rows // GRP, GRP).astype(jnp.int32)
    records = _make_records()(table.T)
    out = _make_gather(n_rows)(x_flat, records)
    return out.reshape(b, s, 128)[:, :, :EMBED_DIM]


# speedup vs baseline: 1.3192x; 1.0445x over previous
import functools

import jax
import jax.numpy as jnp
from jax import lax
from jax.experimental import pallas as pl
from jax.experimental.pallas import tpu as pltpu
from jax.experimental.pallas import tpu_sc as plsc

VOCAB = 1000000
EMBED_DIM = 64
NC, NS = 2, 16
NW = NC * NS
GRP = 128
NBUF = 4
TBLK = 8192
NBLK = -(-VOCAB // TBLK)
REC_ROWS = NBLK * TBLK


def _make_records():
    def rec(tab_ref, out_ref):
        i0 = lax.broadcasted_iota(jnp.int32, (EMBED_DIM, 128), 0)
        i1 = lax.broadcasted_iota(jnp.int32, (EMBED_DIM, 128), 1)
        eye = (i0 == i1).astype(jnp.float32)
        out_ref[...] = lax.dot_general(
            tab_ref[...], eye, (((0,), (0,)), ((), ())),
            preferred_element_type=jnp.float32,
            precision=lax.Precision.HIGHEST,
        )

    return pl.pallas_call(
        rec,
        grid=(NBLK,),
        in_specs=[pl.BlockSpec((EMBED_DIM, TBLK), lambda i: (0, i))],
        out_specs=pl.BlockSpec((TBLK, 128), lambda i: (i, 0)),
        out_shape=jax.ShapeDtypeStruct((REC_ROWS, 128), jnp.float32),
    )


def _make_gather(n_rows):
    rows_per_w = n_rows // NW
    grps_per_w = rows_per_w // GRP
    n_iter = grps_per_w // NBUF
    assert n_rows % (NW * GRP) == 0 and grps_per_w % NBUF == 0

    mesh = plsc.VectorSubcoreMesh(core_axis_name="c", subcore_axis_name="s")

    @functools.partial(
        pl.kernel,
        out_type=jax.ShapeDtypeStruct((n_rows, 128), jnp.float32),
        mesh=mesh,
        scratch_types=[
            pltpu.VMEM((grps_per_w, GRP), jnp.int32),
            pltpu.VMEM((NBUF, GRP, 128), jnp.float32),
            pltpu.SemaphoreType.DMA,
            pltpu.SemaphoreType.DMA,
        ],
    )
    def emb(x_hbm, rec_hbm, out_hbm, idx_v, rows_v, gsem, osem):
        wid = lax.axis_index("s") * NC + lax.axis_index("c")
        grp0 = wid * grps_per_w

        pltpu.sync_copy(x_hbm.at[pl.ds(grp0, grps_per_w)], idx_v)

        def gcopy(g, j):
            return pltpu.make_async_copy(
                rec_hbm.at[idx_v.at[g]], rows_v.at[j], gsem
            )

        def ocopy(g, j):
            return pltpu.make_async_copy(
                rows_v.at[j], out_hbm.at[pl.ds((grp0 + g) * GRP, GRP)], osem
            )

        def body(i, carry):
            base = i * NBUF
            for j in range(NBUF):
                gcopy(base + j, j).start()
            for j in range(NBUF):
                gcopy(base + j, j).wait()
            for j in range(NBUF):
                ocopy(base + j, j).start()
            for j in range(NBUF):
                ocopy(base + j, j).wait()
            return carry

        lax.fori_loop(0, n_iter, body, 0)

    return emb


def kernel(x, table):
    b, s = x.shape
    n_rows = b * s
    x_flat = x.reshape(n_rows // GRP, GRP).astype(jnp.int32)
    records = _make_records()(table.T)
    out = _make_gather(n_rows)(x_flat, records)
    return out.reshape(b, s, 128)[:, :, :EMBED_DIM]


# speedup vs baseline: 1.3415x; 1.0169x over previous
import functools

import jax
import jax.numpy as jnp
from jax import lax
from jax.experimental import pallas as pl
from jax.experimental.pallas import tpu as pltpu
from jax.experimental.pallas import tpu_sc as plsc

VOCAB = 1000000
EMBED_DIM = 64
NC, NS = 2, 16
NW = NC * NS
GRP = 128
NBUF = 5
TBLK = 12288
NBLK = -(-VOCAB // TBLK)
REC_ROWS = NBLK * TBLK


def _make_records():
    def rec(tab_ref, out_ref):
        i0 = lax.broadcasted_iota(jnp.int32, (EMBED_DIM, 128), 0)
        i1 = lax.broadcasted_iota(jnp.int32, (EMBED_DIM, 128), 1)
        eye = (i0 == i1).astype(jnp.float32)
        out_ref[...] = lax.dot_general(
            tab_ref[...], eye, (((0,), (0,)), ((), ())),
            preferred_element_type=jnp.float32,
            precision=lax.Precision.HIGHEST,
        )

    return pl.pallas_call(
        rec,
        grid=(NBLK,),
        in_specs=[pl.BlockSpec((EMBED_DIM, TBLK), lambda i: (0, i))],
        out_specs=pl.BlockSpec((TBLK, 128), lambda i: (i, 0)),
        out_shape=jax.ShapeDtypeStruct((REC_ROWS, 128), jnp.float32),
    )


def _make_gather(n_rows):
    rows_per_w = n_rows // NW
    grps_per_w = rows_per_w // GRP
    n_iter = grps_per_w // NBUF
    assert n_rows % (NW * GRP) == 0 and grps_per_w % NBUF == 0

    mesh = plsc.VectorSubcoreMesh(core_axis_name="c", subcore_axis_name="s")

    @functools.partial(
        pl.kernel,
        out_type=jax.ShapeDtypeStruct((n_rows, 128), jnp.float32),
        mesh=mesh,
        scratch_types=[
            pltpu.VMEM((grps_per_w, GRP), jnp.int32),
            pltpu.VMEM((NBUF, GRP, 128), jnp.float32),
            pltpu.SemaphoreType.DMA,
            pltpu.SemaphoreType.DMA,
        ],
    )
    def emb(x_hbm, rec_hbm, out_hbm, idx_v, rows_v, gsem, osem):
        wid = lax.axis_index("s") * NC + lax.axis_index("c")
        grp0 = wid * grps_per_w

        pltpu.sync_copy(x_hbm.at[pl.ds(grp0, grps_per_w)], idx_v)

        def gcopy(g, j):
            return pltpu.make_async_copy(
                rec_hbm.at[idx_v.at[g]], rows_v.at[j], gsem
            )

        def ocopy(g, j):
            return pltpu.make_async_copy(
                rows_v.at[j], out_hbm.at[pl.ds((grp0 + g) * GRP, GRP)], osem
            )

        def body(i, carry):
            base = i * NBUF
            for j in range(NBUF):
                gcopy(base + j, j).start()
            for j in range(NBUF):
                gcopy(base + j, j).wait()
            for j in range(NBUF):
                ocopy(base + j, j).start()
            for j in range(NBUF):
                ocopy(base + j, j).wait()
            return carry

        lax.fori_loop(0, n_iter, body, 0)

    return emb


def kernel(x, table):
    b, s = x.shape
    n_rows = b * s
    x_flat = x.reshape(n_rows // GRP, GRP).astype(jnp.int32)
    records = _make_records()(table.T)
    out = _make_gather(n_rows)(x_flat, records)
    return out.reshape(b, s, 128)[:, :, :EMBED_DIM]
